# baseline (speedup 1.0000x reference)
import functools

import jax
import jax.numpy as jnp
from jax import lax
from jax.experimental import pallas as pl
from jax.experimental.pallas import tpu as pltpu
from jax.experimental.pallas import tpu_sc as plsc

EMBED_DIM = 64
NC = 2
NS = 16
NW = NC * NS
BLK = 128
LPC = 2
CH = LPC * BLK
NIN = 4
NBUF = 2


def _sc_body(ct_hbm, pt_hbm, ci_hbm, pi_hbm, oc_hbm, x_hbm, y_hbm, w_hbm,
             out_hbm,
             ci_v, pi_v, oc_v, pi2_v, x_v, y_v, rc_v, rp_v, o_v, w_v,
             sem_a, sem_g, sem_s):
    L, B = ci_hbm.shape
    n_steps = L // LPC
    wid = lax.axis_index("s") * NC + lax.axis_index("c")
    sb = pl.ds(wid * BLK, BLK)

    pltpu.sync_copy(w_hbm, w_v)
    w0 = [w_v[0, pl.ds(16 * j, 16)] for j in range(4)]
    w1 = [w_v[1, pl.ds(16 * j, 16)] for j in range(4)]

    def fire_a(ck, slot):
        sl = pl.ds(LPC * ck, LPC)
        pltpu.async_copy(ci_hbm.at[sl, sb], ci_v.at[slot], sem_a[slot])
        pltpu.async_copy(pi_hbm.at[sl, sb], pi_v.at[slot], sem_a[slot])
        pltpu.async_copy(oc_hbm.at[sl, sb], oc_v.at[slot], sem_a[slot])
        pltpu.async_copy(x_hbm.at[sl, sb], x_v.at[slot], sem_a[slot])
        pltpu.async_copy(y_hbm.at[sl, sb], y_v.at[slot], sem_a[slot])

    def wait_a(slot):
        sl = pl.ds(0, LPC)
        pltpu.make_async_copy(ci_hbm.at[sl, sb], ci_v.at[slot], sem_a[slot]).wait()
        pltpu.make_async_copy(pi_hbm.at[sl, sb], pi_v.at[slot], sem_a[slot]).wait()
        pltpu.make_async_copy(oc_hbm.at[sl, sb], oc_v.at[slot], sem_a[slot]).wait()
        pltpu.make_async_copy(x_hbm.at[sl, sb], x_v.at[slot], sem_a[slot]).wait()
        pltpu.make_async_copy(y_hbm.at[sl, sb], y_v.at[slot], sem_a[slot]).wait()

    def compute_idx(slot):
        for h in range(LPC):
            for m in range(BLK // 16):
                s = pl.ds(16 * m, 16)
                pi2_v[slot, h, s] = pi_v[slot, h, s] * 2 + oc_v[slot, h, s]

    def fire_g(slot, p):
        for h in range(LPC):
            pltpu.async_copy(ct_hbm.at[ci_v.at[slot, h]],
                             rc_v.at[p, pl.ds(BLK * h, BLK)], sem_g[p])
            pltpu.async_copy(pt_hbm.at[pi2_v.at[slot, h]],
                             rp_v.at[p, pl.ds(BLK * h, BLK)], sem_g[p])

    def wait_g(slot, p):
        for h in range(LPC):
            pltpu.make_async_copy(ct_hbm.at[ci_v.at[slot, h]],
                                  rc_v.at[p, pl.ds(BLK * h, BLK)], sem_g[p]).wait()
            pltpu.make_async_copy(pt_hbm.at[pi2_v.at[slot, h]],
                                  rp_v.at[p, pl.ds(BLK * h, BLK)], sem_g[p]).wait()

    def fire_s(ck, p):
        pltpu.async_copy(o_v.at[p], out_hbm.at[ck, wid], sem_s[p])

    def wait_s(p):
        pltpu.make_async_copy(o_v.at[p], out_hbm.at[0, wid], sem_s[p]).wait()

    def compute(slot, p):
        def grp_body(h):
            def body(g, tc):
                sg = pl.ds(16 * g, 16)
                xg = x_v[slot, h, sg]
                yg = y_v[slot, h, sg]
                for i in range(16):
                    xs = xg[i]
                    ys = yg[i]
                    t = BLK * h + 16 * g + i
                    for j in range(4):
                        s = pl.ds(16 * j, 16)
                        o_v[p, t, s] = ((rc_v[p, t, s] + rp_v[p, t, s])
                                        + (xs * w0[j] + ys * w1[j]))
                return tc
            return body

        for h in range(LPC):
            lax.fori_loop(0, BLK // 16, grp_body(h), 0)

    for u in range(NIN):
        fire_a(u, u)
    wait_a(0)
    compute_idx(0)
    fire_g(0, 0)

    def quad_body(g2, carry):
        for u in range(NIN):
            ck = g2 * NIN + u
            slot = u
            p = u & 1
            q = 1 - p
            nslot = (u + 1) % NIN

            @pl.when(ck + 1 < n_steps)
            def _():
                wait_a(nslot)
                compute_idx(nslot)
                fire_g(nslot, q)

            wait_g(slot, p)

            @pl.when(ck >= NBUF)
            def _():
                wait_s(p)

            compute(slot, p)
            fire_s(ck, p)

            @pl.when(ck + NIN < n_steps)
            def _():
                fire_a(ck + NIN, slot)
        return carry

    lax.fori_loop(0, n_steps // NIN, quad_body, 0)
    wait_s(0)
    wait_s(1)


def kernel(contour_tensor, point_tensor, location_tensor, on_curve_tensor,
           contour_table, point_table, oncurve_table, loc_W, loc_b):
    B, L = contour_tensor.shape
    scale = float(EMBED_DIM) ** 0.5

    ct2 = contour_table[1:] * scale
    pt2 = ((point_table[1:, None, :] + oncurve_table[None, 1:3, :]
            + loc_b[None, None, :]) * scale).reshape(-1, EMBED_DIM)
    w8 = loc_W.T * scale

    ci = contour_tensor.T.astype(jnp.int32)
    pi = point_tensor.T.astype(jnp.int32)
    oc = on_curve_tensor.T.astype(jnp.int32)
    x = location_tensor[..., 0].T
    y = location_tensor[..., 1].T

    mesh = plsc.VectorSubcoreMesh(core_axis_name="c", subcore_axis_name="s")
    run = functools.partial(
        pl.kernel,
        mesh=mesh,
        compiler_params=pltpu.CompilerParams(use_tc_tiling_on_sc=False,
                                             needs_layout_passes=False),
        out_type=jax.ShapeDtypeStruct((L // LPC, NW, CH, EMBED_DIM), jnp.float32),
        scratch_types=[
            pltpu.VMEM((NIN, LPC, BLK), jnp.int32),
            pltpu.VMEM((NIN, LPC, BLK), jnp.int32),
            pltpu.VMEM((NIN, LPC, BLK), jnp.int32),
            pltpu.VMEM((NIN, LPC, BLK), jnp.int32),
            pltpu.VMEM((NIN, LPC, BLK), jnp.float32),
            pltpu.VMEM((NIN, LPC, BLK), jnp.float32),
            pltpu.VMEM((NBUF, CH, EMBED_DIM), jnp.float32),
            pltpu.VMEM((NBUF, CH, EMBED_DIM), jnp.float32),
            pltpu.VMEM((NBUF, CH, EMBED_DIM), jnp.float32),
            pltpu.VMEM((2, EMBED_DIM), jnp.float32),
            [pltpu.SemaphoreType.DMA] * NIN,
            [pltpu.SemaphoreType.DMA] * NBUF,
            [pltpu.SemaphoreType.DMA] * NBUF,
        ],
    )(_sc_body)
    out = run(ct2, pt2, ci, pi, oc, x, y, w8)
    out = out.reshape(L // LPC, NW, LPC, BLK, EMBED_DIM)
    return out.transpose(1, 3, 0, 2, 4).reshape(B, L, EMBED_DIM)


# speedup vs baseline: 1.2227x; 1.2227x over previous
import functools

import jax
import jax.numpy as jnp
from jax import lax
from jax.experimental import pallas as pl
from jax.experimental.pallas import tpu as pltpu
from jax.experimental.pallas import tpu_sc as plsc

EMBED_DIM = 64
NC = 2
NS = 16
NW = NC * NS
BLK = 128
NIN = 4
NBUF = 2


def _sc_body(ct_hbm, pt_hbm, ci_hbm, pi_hbm, oc_hbm, x_hbm, y_hbm, w_hbm,
             out_hbm,
             ci_v, pi_v, oc_v, pi2_v, x_v, y_v, rc_v, rp_v, o_v, w_v,
             sem_a, sem_g, sem_s):
    L, B = ci_hbm.shape
    wid = lax.axis_index("s") * NC + lax.axis_index("c")
    bb = wid * BLK
    sb = pl.ds(bb, BLK)

    pltpu.sync_copy(w_hbm, w_v)
    w0 = [w_v[0, pl.ds(16 * j, 16)] for j in range(4)]
    w1 = [w_v[1, pl.ds(16 * j, 16)] for j in range(4)]

    def fire_a(ck, slot):
        pltpu.async_copy(ci_hbm.at[ck, sb], ci_v.at[slot], sem_a[slot])
        pltpu.async_copy(pi_hbm.at[ck, sb], pi_v.at[slot], sem_a[slot])
        pltpu.async_copy(oc_hbm.at[ck, sb], oc_v.at[slot], sem_a[slot])
        pltpu.async_copy(x_hbm.at[ck, sb], x_v.at[slot], sem_a[slot])
        pltpu.async_copy(y_hbm.at[ck, sb], y_v.at[slot], sem_a[slot])

    def wait_a(slot):
        pltpu.make_async_copy(ci_hbm.at[0, sb], ci_v.at[slot], sem_a[slot]).wait()
        pltpu.make_async_copy(pi_hbm.at[0, sb], pi_v.at[slot], sem_a[slot]).wait()
        pltpu.make_async_copy(oc_hbm.at[0, sb], oc_v.at[slot], sem_a[slot]).wait()
        pltpu.make_async_copy(x_hbm.at[0, sb], x_v.at[slot], sem_a[slot]).wait()
        pltpu.make_async_copy(y_hbm.at[0, sb], y_v.at[slot], sem_a[slot]).wait()

    def compute_idx(slot):
        for m in range(BLK // 16):
            s = pl.ds(16 * m, 16)
            pi2_v[slot, s] = pi_v[slot, s] * 2 + oc_v[slot, s]

    def fire_g(slot, p):
        pltpu.async_copy(ct_hbm.at[ci_v.at[slot]], rc_v.at[p], sem_g[p])
        pltpu.async_copy(pt_hbm.at[pi2_v.at[slot]], rp_v.at[p], sem_g[p])

    def wait_g(slot, p):
        pltpu.make_async_copy(ct_hbm.at[ci_v.at[slot]], rc_v.at[p], sem_g[p]).wait()
        pltpu.make_async_copy(pt_hbm.at[pi2_v.at[slot]], rp_v.at[p], sem_g[p]).wait()

    def fire_s(ck, p):
        pltpu.async_copy(o_v.at[p], out_hbm.at[ck, wid], sem_s[p])

    def wait_s(p):
        pltpu.make_async_copy(o_v.at[p], out_hbm.at[0, wid], sem_s[p]).wait()

    def compute(slot, p):
        def grp_body(g, tc):
            sg = pl.ds(16 * g, 16)
            xg = x_v[slot, sg]
            yg = y_v[slot, sg]
            for i in range(16):
                xs = xg[i]
                ys = yg[i]
                t = 16 * g + i
                for j in range(4):
                    s = pl.ds(16 * j, 16)
                    o_v[p, t, s] = ((rc_v[p, t, s] + rp_v[p, t, s])
                                    + (xs * w0[j] + ys * w1[j]))
            return tc

        lax.fori_loop(0, BLK // 16, grp_body, 0)

    for u in range(NIN):
        fire_a(u, u)
    wait_a(0)
    compute_idx(0)
    fire_g(0, 0)

    def quad_body(g2, carry):
        for u in range(NIN):
            ck = g2 * NIN + u
            slot = u
            p = u & 1
            q = 1 - p
            nslot = (u + 1) % NIN

            @pl.when(ck + 1 < L)
            def _():
                wait_a(nslot)
                compute_idx(nslot)
                fire_g(nslot, q)

            wait_g(slot, p)

            @pl.when(ck >= NBUF)
            def _():
                wait_s(p)

            compute(slot, p)
            fire_s(ck, p)

            @pl.when(ck + NIN < L)
            def _():
                fire_a(ck + NIN, slot)
        return carry

    lax.fori_loop(0, L // NIN, quad_body, 0)
    wait_s(0)
    wait_s(1)


def kernel(contour_tensor, point_tensor, location_tensor, on_curve_tensor,
           contour_table, point_table, oncurve_table, loc_W, loc_b):
    B, L = contour_tensor.shape
    scale = float(EMBED_DIM) ** 0.5

    ct2 = contour_table[1:] * scale
    pt2 = ((point_table[1:, None, :] + oncurve_table[None, 1:3, :]
            + loc_b[None, None, :]) * scale).reshape(-1, EMBED_DIM)
    w8 = loc_W.T * scale

    ci = contour_tensor.T.astype(jnp.int32)
    pi = point_tensor.T.astype(jnp.int32)
    oc = on_curve_tensor.T.astype(jnp.int32)
    x = location_tensor[..., 0].T
    y = location_tensor[..., 1].T

    mesh = plsc.VectorSubcoreMesh(core_axis_name="c", subcore_axis_name="s")
    run = functools.partial(
        pl.kernel,
        mesh=mesh,
        compiler_params=pltpu.CompilerParams(use_tc_tiling_on_sc=False,
                                             needs_layout_passes=False),
        out_type=jax.ShapeDtypeStruct((L, NW, BLK, EMBED_DIM), jnp.float32),
        scratch_types=[
            pltpu.VMEM((NIN, BLK), jnp.int32),
            pltpu.VMEM((NIN, BLK), jnp.int32),
            pltpu.VMEM((NIN, BLK), jnp.int32),
            pltpu.VMEM((NIN, BLK), jnp.int32),
            pltpu.VMEM((NIN, BLK), jnp.float32),
            pltpu.VMEM((NIN, BLK), jnp.float32),
            pltpu.VMEM((NBUF, BLK, EMBED_DIM), jnp.float32),
            pltpu.VMEM((NBUF, BLK, EMBED_DIM), jnp.float32),
            pltpu.VMEM((NBUF, BLK, EMBED_DIM), jnp.float32),
            pltpu.VMEM((2, EMBED_DIM), jnp.float32),
            [pltpu.SemaphoreType.DMA] * NIN,
            [pltpu.SemaphoreType.DMA] * NBUF,
            [pltpu.SemaphoreType.DMA] * NBUF,
        ],
    )(_sc_body)
    out = run(ct2, pt2, ci, pi, oc, x, y, w8)
    return out.transpose(1, 2, 0, 3).reshape(B, L, EMBED_DIM)
